# baseline (speedup 1.0000x reference)
import jax
import jax.numpy as jnp
from jax.experimental import pallas as pl
from jax.experimental.pallas import tpu as pltpu

_G = 64
_V = 64
_C = 8
_K = 64
_RB = 128


def _fused_kernel(w_ref, ind_ref, xt_ref, out_ref, w2t_ref):
    c = pl.program_id(0)
    i = pl.program_id(1)

    @pl.when(jnp.logical_and(c == 0, i == 0))
    def _build_w2t():
        w = w_ref[...]
        ind = ind_ref[...]
        kk = jax.lax.broadcasted_iota(jnp.int32, (_G, _C, _K), 2)
        m = (ind[:, :, None] == kk).astype(jnp.float32)
        w2t = jax.lax.dot_general(
            w, m, (((2,), (1,)), ((0,), (0,))),
            preferred_element_type=jnp.float32,
        )
        w2t_ref[...] = w2t.reshape(_G * _V, _K)

    out_ref[...] = jnp.dot(
        w2t_ref[pl.ds(i * _RB, _RB), :], xt_ref[...],
        preferred_element_type=jnp.float32,
    )


@jax.jit
def kernel(x, w, ind):
    B = x.shape[0]
    xt = x.T

    bc = B // 2
    outt = pl.pallas_call(
        _fused_kernel,
        grid=(2, _G * _V // _RB),
        in_specs=[
            pl.BlockSpec((_G, _V, _C), lambda c, r: (0, 0, 0)),
            pl.BlockSpec((_G, _C), lambda c, r: (0, 0)),
            pl.BlockSpec((_K, bc), lambda c, r: (0, c)),
        ],
        out_specs=pl.BlockSpec((_RB, bc), lambda c, r: (r, c)),
        out_shape=jax.ShapeDtypeStruct((_G * _V, B), jnp.float32),
        scratch_shapes=[pltpu.VMEM((_G * _V, _K), jnp.float32)],
    )(w, ind, xt)
    return outt.reshape(_G, _V, B).transpose(2, 0, 1)
